# baseline (speedup 1.0000x reference)
import jax
import jax.numpy as jnp
from jax.experimental import pallas as pl
from jax.experimental.pallas import tpu as pltpu

_RH, _RW = 64, 64
_RPS = 8


def _body(idx_ref, top_ref, bot_ref, wt_ref, wb_ref,
          *refs):
    row_blks = refs[:2 * _RPS]
    wxt_blk = refs[2 * _RPS]
    out_blk = refs[2 * _RPS + 1]
    b = pl.program_id(0)
    chunk = pl.program_id(1)
    wxt = wxt_blk[0]
    for i in range(_RPS):
        ry = chunk * _RPS + i
        wt = wt_ref[b, ry]
        wb = wb_ref[b, ry]
        top = row_blks[2 * i][0, :, 0, 0, :]
        bot = row_blks[2 * i + 1][0, :, 0, 0, :]
        tr = jnp.dot(top, wxt, preferred_element_type=jnp.float32)
        br = jnp.dot(bot, wxt, preferred_element_type=jnp.float32)
        out_blk[0, :, ry, :] = tr * wt + br * wb


def kernel(image, boxes, box_indices):
    N, C, H, W = image.shape
    B = boxes.shape[0]

    y1 = boxes[:, 0]
    x1 = boxes[:, 1]
    y2 = boxes[:, 2]
    x2 = boxes[:, 3]
    hs = (y2 - y1) * (H - 1) / float(_RH - 1)
    ws = (x2 - x1) * (W - 1) / float(_RW - 1)
    ty = jnp.arange(_RH, dtype=jnp.float32)
    tx = jnp.arange(_RW, dtype=jnp.float32)
    in_y = y1[:, None] * (H - 1) + ty[None, :] * hs[:, None]
    in_x = x1[:, None] * (W - 1) + tx[None, :] * ws[:, None]
    yvalid = ((in_y >= 0) & (in_y <= H - 1)).astype(jnp.float32)
    xvalid = ((in_x >= 0) & (in_x <= W - 1)).astype(jnp.float32)
    in_y = jnp.where(yvalid > 0, in_y, 0.0)
    in_x = jnp.where(xvalid > 0, in_x, 0.0)
    top_y = jnp.floor(in_y)
    bot_y = jnp.ceil(in_y)
    left_x = jnp.floor(in_x)
    right_x = jnp.ceil(in_x)
    y_l = in_y - top_y
    x_l = in_x - left_x
    w_top = (1.0 - y_l) * yvalid
    w_bot = y_l * yvalid

    cols = jnp.arange(W, dtype=jnp.float32)[None, :, None]
    wxt = ((cols == left_x[:, None, :]) * (1.0 - x_l)[:, None, :]
           + (cols == right_x[:, None, :]) * x_l[:, None, :])
    wxt = (wxt * xvalid[:, None, :]).astype(jnp.float32)

    top_i = top_y.astype(jnp.int32)
    bot_i = bot_y.astype(jnp.int32)

    image5 = image.reshape(N, C, H, 1, W)

    row_specs = []
    for i in range(_RPS):
        def _top_map(b, ch, idx, top, bot, wt, wb, _i=i):
            return (idx[b], 0, top[b, ch * _RPS + _i], 0, 0)

        def _bot_map(b, ch, idx, top, bot, wt, wb, _i=i):
            return (idx[b], 0, bot[b, ch * _RPS + _i], 0, 0)

        row_specs.append(pl.BlockSpec((1, C, 1, 1, W), _top_map))
        row_specs.append(pl.BlockSpec((1, C, 1, 1, W), _bot_map))

    grid_spec = pltpu.PrefetchScalarGridSpec(
        num_scalar_prefetch=5,
        grid=(B, _RH // _RPS),
        in_specs=row_specs + [
            pl.BlockSpec(
                (1, W, _RW),
                lambda b, ch, idx, top, bot, wt, wb: (b, 0, 0)),
        ],
        out_specs=pl.BlockSpec(
            (1, C, _RH, _RW),
            lambda b, ch, idx, top, bot, wt, wb: (b, 0, 0, 0)),
    )

    return pl.pallas_call(
        _body,
        grid_spec=grid_spec,
        out_shape=jax.ShapeDtypeStruct((B, C, _RH, _RW), jnp.float32),
    )(box_indices.astype(jnp.int32), top_i, bot_i, w_top, w_bot,
      *([image5] * (2 * _RPS)), wxt)


# speedup vs baseline: 1.0995x; 1.0995x over previous
import jax
import jax.numpy as jnp
from jax.experimental import pallas as pl
from jax.experimental.pallas import tpu as pltpu

_RH, _RW = 64, 64
_RPS = 16


def _body(idx_ref, top_ref, bot_ref, wt_ref, wb_ref,
          *refs):
    row_blks = refs[:2 * _RPS]
    wxt_blk = refs[2 * _RPS]
    out_blk = refs[2 * _RPS + 1]
    b = pl.program_id(0)
    chunk = pl.program_id(1)
    wxt = wxt_blk[0]
    for i in range(_RPS):
        ry = chunk * _RPS + i
        wt = wt_ref[b, ry]
        wb = wb_ref[b, ry]
        top = row_blks[2 * i][0, :, 0, 0, :]
        bot = row_blks[2 * i + 1][0, :, 0, 0, :]
        tr = jnp.dot(top, wxt, preferred_element_type=jnp.float32)
        br = jnp.dot(bot, wxt, preferred_element_type=jnp.float32)
        out_blk[0, :, ry, :] = tr * wt + br * wb


def kernel(image, boxes, box_indices):
    N, C, H, W = image.shape
    B = boxes.shape[0]

    y1 = boxes[:, 0]
    x1 = boxes[:, 1]
    y2 = boxes[:, 2]
    x2 = boxes[:, 3]
    hs = (y2 - y1) * (H - 1) / float(_RH - 1)
    ws = (x2 - x1) * (W - 1) / float(_RW - 1)
    ty = jnp.arange(_RH, dtype=jnp.float32)
    tx = jnp.arange(_RW, dtype=jnp.float32)
    in_y = y1[:, None] * (H - 1) + ty[None, :] * hs[:, None]
    in_x = x1[:, None] * (W - 1) + tx[None, :] * ws[:, None]
    yvalid = ((in_y >= 0) & (in_y <= H - 1)).astype(jnp.float32)
    xvalid = ((in_x >= 0) & (in_x <= W - 1)).astype(jnp.float32)
    in_y = jnp.where(yvalid > 0, in_y, 0.0)
    in_x = jnp.where(xvalid > 0, in_x, 0.0)
    top_y = jnp.floor(in_y)
    bot_y = jnp.ceil(in_y)
    left_x = jnp.floor(in_x)
    right_x = jnp.ceil(in_x)
    y_l = in_y - top_y
    x_l = in_x - left_x
    w_top = (1.0 - y_l) * yvalid
    w_bot = y_l * yvalid

    cols = jnp.arange(W, dtype=jnp.float32)[None, :, None]
    wxt = ((cols == left_x[:, None, :]) * (1.0 - x_l)[:, None, :]
           + (cols == right_x[:, None, :]) * x_l[:, None, :])
    wxt = (wxt * xvalid[:, None, :]).astype(jnp.float32)

    top_i = top_y.astype(jnp.int32)
    bot_i = bot_y.astype(jnp.int32)

    image5 = image.reshape(N, C, H, 1, W)

    row_specs = []
    for i in range(_RPS):
        def _top_map(b, ch, idx, top, bot, wt, wb, _i=i):
            return (idx[b], 0, top[b, ch * _RPS + _i], 0, 0)

        def _bot_map(b, ch, idx, top, bot, wt, wb, _i=i):
            return (idx[b], 0, bot[b, ch * _RPS + _i], 0, 0)

        row_specs.append(pl.BlockSpec((1, C, 1, 1, W), _top_map))
        row_specs.append(pl.BlockSpec((1, C, 1, 1, W), _bot_map))

    grid_spec = pltpu.PrefetchScalarGridSpec(
        num_scalar_prefetch=5,
        grid=(B, _RH // _RPS),
        in_specs=row_specs + [
            pl.BlockSpec(
                (1, W, _RW),
                lambda b, ch, idx, top, bot, wt, wb: (b, 0, 0)),
        ],
        out_specs=pl.BlockSpec(
            (1, C, _RH, _RW),
            lambda b, ch, idx, top, bot, wt, wb: (b, 0, 0, 0)),
    )

    return pl.pallas_call(
        _body,
        grid_spec=grid_spec,
        out_shape=jax.ShapeDtypeStruct((B, C, _RH, _RW), jnp.float32),
    )(box_indices.astype(jnp.int32), top_i, bot_i, w_top, w_bot,
      *([image5] * (2 * _RPS)), wxt)


# speedup vs baseline: 1.2106x; 1.1011x over previous
import jax
import jax.numpy as jnp
from jax.experimental import pallas as pl
from jax.experimental.pallas import tpu as pltpu

_RH, _RW = 64, 64
_RPS = 16


def _body(idx_ref, top_ref, bot_ref, wt_ref, wb_ref,
          *refs):
    row_blks = refs[:2 * _RPS]
    wxt_blk = refs[2 * _RPS]
    out_blk = refs[2 * _RPS + 1]
    b = pl.program_id(0)
    chunk = pl.program_id(1)
    wxt = wxt_blk[0]
    for i in range(_RPS):
        ry = chunk * _RPS + i
        wt = wt_ref[b, ry]
        wb = wb_ref[b, ry]
        top = row_blks[2 * i][0, 0]
        bot = row_blks[2 * i + 1][0, 0]
        tr = jnp.dot(top, wxt, preferred_element_type=jnp.float32)
        br = jnp.dot(bot, wxt, preferred_element_type=jnp.float32)
        out_blk[0, :, ry, :] = tr * wt + br * wb


def kernel(image, boxes, box_indices):
    N, C, H, W = image.shape
    B = boxes.shape[0]

    y1 = boxes[:, 0]
    x1 = boxes[:, 1]
    y2 = boxes[:, 2]
    x2 = boxes[:, 3]
    hs = (y2 - y1) * (H - 1) / float(_RH - 1)
    ws = (x2 - x1) * (W - 1) / float(_RW - 1)
    ty = jnp.arange(_RH, dtype=jnp.float32)
    tx = jnp.arange(_RW, dtype=jnp.float32)
    in_y = y1[:, None] * (H - 1) + ty[None, :] * hs[:, None]
    in_x = x1[:, None] * (W - 1) + tx[None, :] * ws[:, None]
    yvalid = ((in_y >= 0) & (in_y <= H - 1)).astype(jnp.float32)
    xvalid = ((in_x >= 0) & (in_x <= W - 1)).astype(jnp.float32)
    in_y = jnp.where(yvalid > 0, in_y, 0.0)
    in_x = jnp.where(xvalid > 0, in_x, 0.0)
    top_y = jnp.floor(in_y)
    bot_y = jnp.ceil(in_y)
    left_x = jnp.floor(in_x)
    right_x = jnp.ceil(in_x)
    y_l = in_y - top_y
    x_l = in_x - left_x
    w_top = (1.0 - y_l) * yvalid
    w_bot = y_l * yvalid

    cols = jnp.arange(W, dtype=jnp.float32)[None, :, None]
    wxt = ((cols == left_x[:, None, :]) * (1.0 - x_l)[:, None, :]
           + (cols == right_x[:, None, :]) * x_l[:, None, :])
    wxt = (wxt * xvalid[:, None, :]).astype(jnp.bfloat16)

    top_i = top_y.astype(jnp.int32)
    bot_i = bot_y.astype(jnp.int32)

    image_t = image.transpose(0, 2, 1, 3).astype(jnp.bfloat16)

    row_specs = []
    for i in range(_RPS):
        def _top_map(b, ch, idx, top, bot, wt, wb, _i=i):
            return (idx[b], top[b, ch * _RPS + _i], 0, 0)

        def _bot_map(b, ch, idx, top, bot, wt, wb, _i=i):
            return (idx[b], bot[b, ch * _RPS + _i], 0, 0)

        row_specs.append(pl.BlockSpec((1, 1, C, W), _top_map))
        row_specs.append(pl.BlockSpec((1, 1, C, W), _bot_map))

    grid_spec = pltpu.PrefetchScalarGridSpec(
        num_scalar_prefetch=5,
        grid=(B, _RH // _RPS),
        in_specs=row_specs + [
            pl.BlockSpec(
                (1, W, _RW),
                lambda b, ch, idx, top, bot, wt, wb: (b, 0, 0)),
        ],
        out_specs=pl.BlockSpec(
            (1, C, _RH, _RW),
            lambda b, ch, idx, top, bot, wt, wb: (b, 0, 0, 0)),
    )

    return pl.pallas_call(
        _body,
        grid_spec=grid_spec,
        out_shape=jax.ShapeDtypeStruct((B, C, _RH, _RW), jnp.float32),
    )(box_indices.astype(jnp.int32), top_i, bot_i, w_top, w_bot,
      *([image_t] * (2 * _RPS)), wxt)


# speedup vs baseline: 1.2761x; 1.0541x over previous
import jax
import jax.numpy as jnp
from jax.experimental import pallas as pl
from jax.experimental.pallas import tpu as pltpu
from jax._src.pallas import core as pl_core

_RH, _RW = 64, 64
_RPS = 16


def _body(idx_ref, start_ref, w0_ref, w1_ref, *refs):
    pair_blks = refs[:_RPS]
    wxt_blk = refs[_RPS]
    out_blk = refs[_RPS + 1]
    b = pl.program_id(0)
    chunk = pl.program_id(1)
    wxt = wxt_blk[0]
    C = out_blk.shape[1]
    for i in range(_RPS):
        ry = chunk * _RPS + i
        pair = pair_blks[i][0]
        m = pair.reshape(2 * C, pair.shape[-1])
        a = jnp.dot(m, wxt, preferred_element_type=jnp.float32)
        out_blk[0, :, ry, :] = a[:C] * w0_ref[b, ry] + a[C:] * w1_ref[b, ry]


def kernel(image, boxes, box_indices):
    N, C, H, W = image.shape
    B = boxes.shape[0]

    y1 = boxes[:, 0]
    x1 = boxes[:, 1]
    y2 = boxes[:, 2]
    x2 = boxes[:, 3]
    hs = (y2 - y1) * (H - 1) / float(_RH - 1)
    ws = (x2 - x1) * (W - 1) / float(_RW - 1)
    ty = jnp.arange(_RH, dtype=jnp.float32)
    tx = jnp.arange(_RW, dtype=jnp.float32)
    in_y = y1[:, None] * (H - 1) + ty[None, :] * hs[:, None]
    in_x = x1[:, None] * (W - 1) + tx[None, :] * ws[:, None]
    yvalid = ((in_y >= 0) & (in_y <= H - 1)).astype(jnp.float32)
    xvalid = ((in_x >= 0) & (in_x <= W - 1)).astype(jnp.float32)
    in_y = jnp.where(yvalid > 0, in_y, 0.0)
    in_x = jnp.where(xvalid > 0, in_x, 0.0)
    top_y = jnp.floor(in_y)
    bot_y = jnp.ceil(in_y)
    left_x = jnp.floor(in_x)
    right_x = jnp.ceil(in_x)
    y_l = in_y - top_y
    x_l = in_x - left_x
    w_top = (1.0 - y_l) * yvalid
    w_bot = y_l * yvalid

    top_i = top_y.astype(jnp.int32)
    bot_i = bot_y.astype(jnp.int32)
    start = jnp.minimum(top_i, H - 2)
    top_at0 = top_i == start
    w0 = jnp.where(top_at0, w_top + jnp.where(bot_i == top_i, w_bot, 0.0), 0.0)
    w1 = jnp.where(top_at0, jnp.where(bot_i == top_i + 1, w_bot, 0.0),
                   w_top + w_bot)

    cols = jnp.arange(W, dtype=jnp.float32)[None, :, None]
    wxt = ((cols == left_x[:, None, :]) * (1.0 - x_l)[:, None, :]
           + (cols == right_x[:, None, :]) * x_l[:, None, :])
    wxt = (wxt * xvalid[:, None, :]).astype(jnp.bfloat16)

    image_t = image.transpose(0, 2, 1, 3).astype(jnp.bfloat16)

    pair_specs = []
    for i in range(_RPS):
        def _map(b, ch, idx, st, w0r, w1r, _i=i):
            return (idx[b], st[b, ch * _RPS + _i], 0, 0)

        pair_specs.append(
            pl.BlockSpec((pl_core.Element(1), pl_core.Element(2),
                          pl_core.Element(C), pl_core.Element(W)), _map))

    grid_spec = pltpu.PrefetchScalarGridSpec(
        num_scalar_prefetch=4,
        grid=(B, _RH // _RPS),
        in_specs=pair_specs + [
            pl.BlockSpec(
                (1, W, _RW),
                lambda b, ch, idx, st, w0r, w1r: (b, 0, 0)),
        ],
        out_specs=pl.BlockSpec(
            (1, C, _RH, _RW),
            lambda b, ch, idx, st, w0r, w1r: (b, 0, 0, 0)),
    )

    return pl.pallas_call(
        _body,
        grid_spec=grid_spec,
        out_shape=jax.ShapeDtypeStruct((B, C, _RH, _RW), jnp.float32),
    )(box_indices.astype(jnp.int32), start, w0, w1,
      *([image_t] * _RPS), wxt)


# speedup vs baseline: 1.3918x; 1.0906x over previous
import jax
import jax.numpy as jnp
from jax.experimental import pallas as pl
from jax.experimental.pallas import tpu as pltpu
from jax._src.pallas import core as pl_core

_RH, _RW = 64, 64
_RPS = 32


def _body(idx_ref, start_ref, w0_ref, w1_ref, *refs):
    pair_blks = refs[:_RPS]
    wxt_blk = refs[_RPS]
    out_blk = refs[_RPS + 1]
    b = pl.program_id(0)
    chunk = pl.program_id(1)
    wxt = wxt_blk[0]
    C = out_blk.shape[1]
    for i in range(_RPS):
        ry = chunk * _RPS + i
        pair = pair_blks[i][0]
        m = pair.reshape(2 * C, pair.shape[-1])
        a = jnp.dot(m, wxt, preferred_element_type=jnp.float32)
        out_blk[0, :, ry, :] = a[:C] * w0_ref[b, ry] + a[C:] * w1_ref[b, ry]


def kernel(image, boxes, box_indices):
    N, C, H, W = image.shape
    B = boxes.shape[0]

    y1 = boxes[:, 0]
    x1 = boxes[:, 1]
    y2 = boxes[:, 2]
    x2 = boxes[:, 3]
    hs = (y2 - y1) * (H - 1) / float(_RH - 1)
    ws = (x2 - x1) * (W - 1) / float(_RW - 1)
    ty = jnp.arange(_RH, dtype=jnp.float32)
    tx = jnp.arange(_RW, dtype=jnp.float32)
    in_y = y1[:, None] * (H - 1) + ty[None, :] * hs[:, None]
    in_x = x1[:, None] * (W - 1) + tx[None, :] * ws[:, None]
    yvalid = ((in_y >= 0) & (in_y <= H - 1)).astype(jnp.float32)
    xvalid = ((in_x >= 0) & (in_x <= W - 1)).astype(jnp.float32)
    in_y = jnp.where(yvalid > 0, in_y, 0.0)
    in_x = jnp.where(xvalid > 0, in_x, 0.0)
    top_y = jnp.floor(in_y)
    bot_y = jnp.ceil(in_y)
    left_x = jnp.floor(in_x)
    right_x = jnp.ceil(in_x)
    y_l = in_y - top_y
    x_l = in_x - left_x
    w_top = (1.0 - y_l) * yvalid
    w_bot = y_l * yvalid

    top_i = top_y.astype(jnp.int32)
    bot_i = bot_y.astype(jnp.int32)
    start = jnp.minimum(top_i, H - 2)
    top_at0 = top_i == start
    w0 = jnp.where(top_at0, w_top + jnp.where(bot_i == top_i, w_bot, 0.0), 0.0)
    w1 = jnp.where(top_at0, jnp.where(bot_i == top_i + 1, w_bot, 0.0),
                   w_top + w_bot)

    cols = jnp.arange(W, dtype=jnp.float32)[None, :, None]
    wxt = ((cols == left_x[:, None, :]) * (1.0 - x_l)[:, None, :]
           + (cols == right_x[:, None, :]) * x_l[:, None, :])
    wxt = (wxt * xvalid[:, None, :]).astype(jnp.bfloat16)

    image_t = image.transpose(0, 2, 1, 3).astype(jnp.bfloat16)

    pair_specs = []
    for i in range(_RPS):
        def _map(b, ch, idx, st, w0r, w1r, _i=i):
            return (idx[b], st[b, ch * _RPS + _i], 0, 0)

        pair_specs.append(
            pl.BlockSpec((pl_core.Element(1), pl_core.Element(2),
                          pl_core.Element(C), pl_core.Element(W)), _map))

    grid_spec = pltpu.PrefetchScalarGridSpec(
        num_scalar_prefetch=4,
        grid=(B, _RH // _RPS),
        in_specs=pair_specs + [
            pl.BlockSpec(
                (1, W, _RW),
                lambda b, ch, idx, st, w0r, w1r: (b, 0, 0)),
        ],
        out_specs=pl.BlockSpec(
            (1, C, _RH, _RW),
            lambda b, ch, idx, st, w0r, w1r: (b, 0, 0, 0)),
    )

    return pl.pallas_call(
        _body,
        grid_spec=grid_spec,
        out_shape=jax.ShapeDtypeStruct((B, C, _RH, _RW), jnp.float32),
    )(box_indices.astype(jnp.int32), start, w0, w1,
      *([image_t] * _RPS), wxt)


# speedup vs baseline: 1.6862x; 1.2115x over previous
import jax
import jax.numpy as jnp
from jax.experimental import pallas as pl
from jax.experimental.pallas import tpu as pltpu
from jax._src.pallas import core as pl_core

_RH, _RW = 64, 64
_RPS = 64


def _body(idx_ref, start_ref, w0_ref, w1_ref, *refs):
    pair_blks = refs[:_RPS]
    wxt_blk = refs[_RPS]
    out_blk = refs[_RPS + 1]
    b = pl.program_id(0)
    chunk = pl.program_id(1)
    wxt = wxt_blk[0]
    C = out_blk.shape[1]
    for i in range(_RPS):
        ry = chunk * _RPS + i
        pair = pair_blks[i][0]
        m = pair.reshape(2 * C, pair.shape[-1])
        a = jnp.dot(m, wxt, preferred_element_type=jnp.float32)
        out_blk[0, :, ry, :] = a[:C] * w0_ref[b, ry] + a[C:] * w1_ref[b, ry]


_HCHUNK = 32


def _relayout_body(img_blk, out_blk):
    out_blk[0] = jnp.transpose(img_blk[0], (1, 0, 2)).astype(jnp.bfloat16)


def _relayout(image):
    N, C, H, W = image.shape
    return pl.pallas_call(
        _relayout_body,
        grid=(N, H // _HCHUNK),
        in_specs=[pl.BlockSpec((1, C, _HCHUNK, W), lambda n, h: (n, 0, h, 0))],
        out_specs=pl.BlockSpec((1, _HCHUNK, C, W), lambda n, h: (n, h, 0, 0)),
        out_shape=jax.ShapeDtypeStruct((N, H, C, W), jnp.bfloat16),
    )(image)


def kernel(image, boxes, box_indices):
    N, C, H, W = image.shape
    B = boxes.shape[0]

    y1 = boxes[:, 0]
    x1 = boxes[:, 1]
    y2 = boxes[:, 2]
    x2 = boxes[:, 3]
    hs = (y2 - y1) * (H - 1) / float(_RH - 1)
    ws = (x2 - x1) * (W - 1) / float(_RW - 1)
    ty = jnp.arange(_RH, dtype=jnp.float32)
    tx = jnp.arange(_RW, dtype=jnp.float32)
    in_y = y1[:, None] * (H - 1) + ty[None, :] * hs[:, None]
    in_x = x1[:, None] * (W - 1) + tx[None, :] * ws[:, None]
    yvalid = ((in_y >= 0) & (in_y <= H - 1)).astype(jnp.float32)
    xvalid = ((in_x >= 0) & (in_x <= W - 1)).astype(jnp.float32)
    in_y = jnp.where(yvalid > 0, in_y, 0.0)
    in_x = jnp.where(xvalid > 0, in_x, 0.0)
    top_y = jnp.floor(in_y)
    bot_y = jnp.ceil(in_y)
    left_x = jnp.floor(in_x)
    right_x = jnp.ceil(in_x)
    y_l = in_y - top_y
    x_l = in_x - left_x
    w_top = (1.0 - y_l) * yvalid
    w_bot = y_l * yvalid

    top_i = top_y.astype(jnp.int32)
    bot_i = bot_y.astype(jnp.int32)
    start = jnp.minimum(top_i, H - 2)
    top_at0 = top_i == start
    w0 = jnp.where(top_at0, w_top + jnp.where(bot_i == top_i, w_bot, 0.0), 0.0)
    w1 = jnp.where(top_at0, jnp.where(bot_i == top_i + 1, w_bot, 0.0),
                   w_top + w_bot)

    cols = jnp.arange(W, dtype=jnp.float32)[None, :, None]
    wxt = ((cols == left_x[:, None, :]) * (1.0 - x_l)[:, None, :]
           + (cols == right_x[:, None, :]) * x_l[:, None, :])
    wxt = (wxt * xvalid[:, None, :]).astype(jnp.bfloat16)

    image_t = _relayout(image)

    pair_specs = []
    for i in range(_RPS):
        def _map(b, ch, idx, st, w0r, w1r, _i=i):
            return (idx[b], st[b, ch * _RPS + _i], 0, 0)

        pair_specs.append(
            pl.BlockSpec((pl_core.Element(1), pl_core.Element(2),
                          pl_core.Element(C), pl_core.Element(W)), _map))

    grid_spec = pltpu.PrefetchScalarGridSpec(
        num_scalar_prefetch=4,
        grid=(B, _RH // _RPS),
        in_specs=pair_specs + [
            pl.BlockSpec(
                (1, W, _RW),
                lambda b, ch, idx, st, w0r, w1r: (b, 0, 0)),
        ],
        out_specs=pl.BlockSpec(
            (1, C, _RH, _RW),
            lambda b, ch, idx, st, w0r, w1r: (b, 0, 0, 0)),
    )

    return pl.pallas_call(
        _body,
        grid_spec=grid_spec,
        out_shape=jax.ShapeDtypeStruct((B, C, _RH, _RW), jnp.float32),
    )(box_indices.astype(jnp.int32), start, w0, w1,
      *([image_t] * _RPS), wxt)


# speedup vs baseline: 1.6990x; 1.0076x over previous
import jax
import jax.numpy as jnp
from jax.experimental import pallas as pl
from jax.experimental.pallas import tpu as pltpu
from jax._src.pallas import core as pl_core

_RH, _RW = 64, 64
_RPS = 64


def _body(idx_ref, start_ref, w0_ref, w1_ref, *refs):
    pair_blks = refs[:_RPS]
    wxt_blk = refs[_RPS]
    out_blk = refs[_RPS + 1]
    b = pl.program_id(0)
    chunk = pl.program_id(1)
    wxt = wxt_blk[0]
    C = out_blk.shape[1]
    for i in range(_RPS):
        ry = chunk * _RPS + i
        pair = pair_blks[i][0]
        m = pair.reshape(2 * C, pair.shape[-1])
        a = jnp.dot(m, wxt, preferred_element_type=jnp.float32)
        out_blk[0, :, ry, :] = a[:C] * w0_ref[b, ry] + a[C:] * w1_ref[b, ry]


_HCHUNK = 64


def _relayout_body(img_blk, out_blk):
    out_blk[0] = jnp.transpose(img_blk[0], (1, 0, 2)).astype(jnp.bfloat16)


def _relayout(image):
    N, C, H, W = image.shape
    return pl.pallas_call(
        _relayout_body,
        grid=(N, H // _HCHUNK),
        in_specs=[pl.BlockSpec((1, C, _HCHUNK, W), lambda n, h: (n, 0, h, 0))],
        out_specs=pl.BlockSpec((1, _HCHUNK, C, W), lambda n, h: (n, h, 0, 0)),
        out_shape=jax.ShapeDtypeStruct((N, H, C, W), jnp.bfloat16),
    )(image)


def kernel(image, boxes, box_indices):
    N, C, H, W = image.shape
    B = boxes.shape[0]

    y1 = boxes[:, 0]
    x1 = boxes[:, 1]
    y2 = boxes[:, 2]
    x2 = boxes[:, 3]
    hs = (y2 - y1) * (H - 1) / float(_RH - 1)
    ws = (x2 - x1) * (W - 1) / float(_RW - 1)
    ty = jnp.arange(_RH, dtype=jnp.float32)
    tx = jnp.arange(_RW, dtype=jnp.float32)
    in_y = y1[:, None] * (H - 1) + ty[None, :] * hs[:, None]
    in_x = x1[:, None] * (W - 1) + tx[None, :] * ws[:, None]
    yvalid = ((in_y >= 0) & (in_y <= H - 1)).astype(jnp.float32)
    xvalid = ((in_x >= 0) & (in_x <= W - 1)).astype(jnp.float32)
    in_y = jnp.where(yvalid > 0, in_y, 0.0)
    in_x = jnp.where(xvalid > 0, in_x, 0.0)
    top_y = jnp.floor(in_y)
    bot_y = jnp.ceil(in_y)
    left_x = jnp.floor(in_x)
    right_x = jnp.ceil(in_x)
    y_l = in_y - top_y
    x_l = in_x - left_x
    w_top = (1.0 - y_l) * yvalid
    w_bot = y_l * yvalid

    top_i = top_y.astype(jnp.int32)
    bot_i = bot_y.astype(jnp.int32)
    start = jnp.minimum(top_i, H - 2)
    top_at0 = top_i == start
    w0 = jnp.where(top_at0, w_top + jnp.where(bot_i == top_i, w_bot, 0.0), 0.0)
    w1 = jnp.where(top_at0, jnp.where(bot_i == top_i + 1, w_bot, 0.0),
                   w_top + w_bot)

    cols = jnp.arange(W, dtype=jnp.float32)[None, :, None]
    wxt = ((cols == left_x[:, None, :]) * (1.0 - x_l)[:, None, :]
           + (cols == right_x[:, None, :]) * x_l[:, None, :])
    wxt = (wxt * xvalid[:, None, :]).astype(jnp.bfloat16)

    image_t = _relayout(image)

    pair_specs = []
    for i in range(_RPS):
        def _map(b, ch, idx, st, w0r, w1r, _i=i):
            return (idx[b], st[b, ch * _RPS + _i], 0, 0)

        pair_specs.append(
            pl.BlockSpec((pl_core.Element(1), pl_core.Element(2),
                          pl_core.Element(C), pl_core.Element(W)), _map))

    grid_spec = pltpu.PrefetchScalarGridSpec(
        num_scalar_prefetch=4,
        grid=(B, _RH // _RPS),
        in_specs=pair_specs + [
            pl.BlockSpec(
                (1, W, _RW),
                lambda b, ch, idx, st, w0r, w1r: (b, 0, 0)),
        ],
        out_specs=pl.BlockSpec(
            (1, C, _RH, _RW),
            lambda b, ch, idx, st, w0r, w1r: (b, 0, 0, 0)),
    )

    return pl.pallas_call(
        _body,
        grid_spec=grid_spec,
        out_shape=jax.ShapeDtypeStruct((B, C, _RH, _RW), jnp.float32),
    )(box_indices.astype(jnp.int32), start, w0, w1,
      *([image_t] * _RPS), wxt)
